# baseline (speedup 1.0000x reference)
import functools

import jax
import jax.numpy as jnp
from jax import lax
from jax.experimental import pallas as pl
from jax.experimental.pallas import tpu as pltpu
from jax.experimental.pallas import tpu_sc as plsc

NTOK = 2048
H = 2048
DFF = 5632
NSLOT = 4

MT = 256
FT = 512
NF = DFF // FT
NM = NTOK // MT

T = 256
NT = NTOK // T + NSLOT
BP = NT * T
AF = 256
NAF = 512 // AF

_NC = 2
_NW = 32
_BW = BP // _NW
_CH = 16
_NCH = _BW // _CH


def _silu(g):
    return g * jax.nn.sigmoid(g)


def _dot_nt(a, b):
    return jax.lax.dot_general(
        a.astype(jnp.bfloat16), b.astype(jnp.bfloat16),
        (((1,), (1,)), ((), ())), preferred_element_type=jnp.float32)


def _base_body(x_ref, gw_ref, uw_ref, dw_ref, out_ref):
    f = pl.program_id(0)
    m = pl.program_id(1)
    xm = x_ref[pl.ds(m * MT, MT), :]
    h = _silu(_dot_nt(xm, gw_ref[...])) * _dot_nt(xm, uw_ref[...])
    contrib = jax.lax.dot_general(
        h.astype(jnp.bfloat16), dw_ref[...].astype(jnp.bfloat16),
        (((1,), (1,)), ((), ())), preferred_element_type=jnp.float32)

    @pl.when(f == 0)
    def _():
        out_ref[pl.ds(m * MT, MT), :] = contrib

    @pl.when(f != 0)
    def _():
        out_ref[pl.ds(m * MT, MT), :] += contrib


def _base_mlp(x_bf, gate_w, up_w, down_w):
    full = pl.BlockSpec((NTOK, H), lambda *_: (0, 0))
    return pl.pallas_call(
        _base_body,
        grid=(NF, NM),
        in_specs=[
            full,
            pl.BlockSpec((FT, H), lambda f, m: (f, 0)),
            pl.BlockSpec((FT, H), lambda f, m: (f, 0)),
            pl.BlockSpec((H, FT), lambda f, m: (0, f)),
        ],
        out_specs=full,
        out_shape=jax.ShapeDtypeStruct((NTOK, H), jnp.float32),
        compiler_params=pltpu.CompilerParams(
            dimension_semantics=("arbitrary", "arbitrary")),
    )(x_bf, gate_w, up_w, down_w)


def _sc_gather_body(x_hbm, base_hbm, gidx_hbm, xg_hbm, bg_hbm,
                    idx_v, xbuf, bbuf, sem1, sem2):
    wid = lax.axis_index("s") * _NC + lax.axis_index("c")
    off = wid * _BW
    pltpu.sync_copy(gidx_hbm.at[pl.ds(off, _BW)], idx_v)
    for c in range(_NCH):
        iv = idx_v.at[pl.ds(c * _CH, _CH)]
        cp1 = pltpu.async_copy(x_hbm.at[iv], xbuf, sem1)
        cp2 = pltpu.async_copy(base_hbm.at[iv], bbuf, sem2)
        cp1.wait()
        cp2.wait()
        pltpu.sync_copy(xbuf, xg_hbm.at[pl.ds(off + c * _CH, _CH)])
        pltpu.sync_copy(bbuf, bg_hbm.at[pl.ds(off + c * _CH, _CH)])


def _gather_rows(x, base_out, gidx):
    mesh = plsc.VectorSubcoreMesh(core_axis_name="c", subcore_axis_name="s")
    fn = functools.partial(
        pl.kernel,
        mesh=mesh,
        out_type=(jax.ShapeDtypeStruct((BP, H), jnp.float32),
                  jax.ShapeDtypeStruct((BP, H), jnp.float32)),
        scratch_types=[
            pltpu.VMEM((_BW,), jnp.int32),
            pltpu.VMEM((_CH, H), jnp.float32),
            pltpu.VMEM((_CH, H), jnp.float32),
            pltpu.SemaphoreType.DMA,
            pltpu.SemaphoreType.DMA,
        ],
    )(_sc_gather_body)
    return fn(x, base_out, gidx)


def _adapter_body(sr_ref, xg_ref, bg_ref, rg_ref, ru_ref, rd_ref,
                  fg_ref, fu_ref, fd_ref, scales_ref, out_ref):
    t = pl.program_id(0)
    f = pl.program_id(1)
    slot = sr_ref[t]
    rs = scales_ref[slot, 0]
    fs = scales_ref[slot, 1]
    xm = xg_ref[...].astype(jnp.bfloat16)

    hr = _silu(_dot_nt(xm, rg_ref[0])) * _dot_nt(xm, ru_ref[0]) * rs
    contrib = jax.lax.dot_general(
        hr.astype(jnp.bfloat16), rd_ref[0].astype(jnp.bfloat16),
        (((1,), (1,)), ((), ())), preferred_element_type=jnp.float32)
    hf = _silu(_dot_nt(xm, fg_ref[0])) * _dot_nt(xm, fu_ref[0]) * fs
    contrib += jax.lax.dot_general(
        hf.astype(jnp.bfloat16), fd_ref[0].astype(jnp.bfloat16),
        (((1,), (1,)), ((), ())), preferred_element_type=jnp.float32)

    @pl.when(f == 0)
    def _():
        out_ref[...] = bg_ref[...] + contrib

    @pl.when(f != 0)
    def _():
        out_ref[...] += contrib


def _adapter_tiles(tile_slot, xg, bg, retain_gate, retain_up, retain_down,
                   forget_gate, forget_up, forget_down, scales):
    grid_spec = pltpu.PrefetchScalarGridSpec(
        num_scalar_prefetch=1,
        grid=(NT, NAF),
        in_specs=[
            pl.BlockSpec((T, H), lambda t, f, sr: (t, 0)),
            pl.BlockSpec((T, H), lambda t, f, sr: (t, 0)),
            pl.BlockSpec((1, AF, H), lambda t, f, sr: (sr[t], f, 0)),
            pl.BlockSpec((1, AF, H), lambda t, f, sr: (sr[t], f, 0)),
            pl.BlockSpec((1, H, AF), lambda t, f, sr: (sr[t], 0, f)),
            pl.BlockSpec((1, AF, H), lambda t, f, sr: (sr[t], f, 0)),
            pl.BlockSpec((1, AF, H), lambda t, f, sr: (sr[t], f, 0)),
            pl.BlockSpec((1, H, AF), lambda t, f, sr: (sr[t], 0, f)),
            pl.BlockSpec(memory_space=pltpu.SMEM),
        ],
        out_specs=pl.BlockSpec((T, H), lambda t, f, sr: (t, 0)),
    )
    return pl.pallas_call(
        _adapter_body,
        grid_spec=grid_spec,
        out_shape=jax.ShapeDtypeStruct((BP, H), jnp.float32),
        compiler_params=pltpu.CompilerParams(
            dimension_semantics=("arbitrary", "arbitrary")),
    )(tile_slot, xg, bg, retain_gate, retain_up, retain_down,
      forget_gate, forget_up, forget_down, scales)


def _sc_scatter_body(ap_hbm, sidx_hbm, out_hbm, idx_v, buf, sem):
    wid = lax.axis_index("s") * _NC + lax.axis_index("c")
    off = wid * _BW
    pltpu.sync_copy(sidx_hbm.at[wid], idx_v)
    for c in range(_NCH):
        pltpu.sync_copy(ap_hbm.at[pl.ds(off + c * _CH, _CH)], buf)
        pltpu.async_copy(buf, out_hbm.at[idx_v.at[c]], sem).wait()


def _scatter_rows(ap, sidx):
    mesh = plsc.VectorSubcoreMesh(core_axis_name="c", subcore_axis_name="s")
    fn = functools.partial(
        pl.kernel,
        mesh=mesh,
        out_type=jax.ShapeDtypeStruct((NTOK + 8, H), jnp.float32),
        scratch_types=[
            pltpu.VMEM((_NCH, _CH), jnp.int32),
            pltpu.VMEM((_CH, H), jnp.float32),
            pltpu.SemaphoreType.DMA,
        ],
    )(_sc_scatter_body)
    return fn(ap, sidx)


def _routing(ti):
    perm = jnp.argsort(ti)
    counts = jnp.bincount(ti, length=NSLOT)
    tiles_per = (counts + T - 1) // T
    tile_bound = jnp.cumsum(tiles_per)
    offs = jnp.concatenate([jnp.zeros(1, jnp.int32),
                            jnp.cumsum(counts)]).astype(jnp.int32)
    tile_start = jnp.concatenate([jnp.zeros(1, jnp.int32),
                                  tile_bound]).astype(jnp.int32)
    tix = jnp.arange(NT, dtype=jnp.int32)
    tile_slot = jnp.clip(
        jnp.searchsorted(tile_bound, tix, side='right'),
        0, NSLOT - 1).astype(jnp.int32)
    p = jnp.arange(BP, dtype=jnp.int32)
    sp = tile_slot[p // T]
    local = p - T * tile_start[sp]
    valid = local < counts[sp]
    src = jnp.clip(offs[sp] + local, 0, NTOK - 1)
    row_ids = jnp.where(valid, perm[src].astype(jnp.int32), NTOK)
    gidx = jnp.minimum(row_ids, NTOK - 1)
    return tile_slot, gidx, row_ids


def kernel(x, token_indices, gate_w, up_w, down_w, retain_gate, retain_up,
           retain_down, forget_gate, forget_up, forget_down, scales):
    ti = token_indices.astype(jnp.int32)
    tile_slot, gidx, row_ids = _routing(ti)

    base_out = _base_mlp(x.astype(jnp.bfloat16), gate_w, up_w, down_w)
    xg, bg = _gather_rows(x, base_out, gidx)
    ap = _adapter_tiles(tile_slot, xg, bg,
                        retain_gate, retain_up, retain_down,
                        forget_gate, forget_up, forget_down, scales)
    out_pad = _scatter_rows(ap, row_ids.reshape(_NW, _NCH, _CH))
    return lax.slice_in_dim(out_pad, 0, NTOK, axis=0)
